# baseline (speedup 1.0000x reference)
import functools

import jax
import jax.numpy as jnp
from jax import lax
from jax.experimental import pallas as pl
from jax.experimental.pallas import tpu as pltpu
from jax.experimental.pallas import tpu_sc as plsc

N_ITEMS = 8192
BATCH_N = 1024
HALF_N = BATCH_N // 2
LAT = 200
TAU_C = 0.2

_NC = 2
_NS = 16
_NW = _NC * _NS
_CH = 8
_HALFROW = N_ITEMS // 2
_NBUF = 3


@functools.cache
def _make_sc_gather(rows):
    bpw = rows // _NW
    n_units = (bpw // _CH) * 2

    @functools.partial(
        pl.kernel,
        mesh=plsc.VectorSubcoreMesh(core_axis_name="c", subcore_axis_name="s"),
        out_type=jax.ShapeDtypeStruct((rows, N_ITEMS), jnp.float32),
        scratch_types=[
            pltpu.VMEM((bpw,), jnp.int32),
        ] + [pltpu.VMEM((_CH, _HALFROW), jnp.float32) for _ in range(_NBUF)]
          + [pltpu.SemaphoreType.DMA for _ in range(_NBUF)],
    )
    def _sc_gather(gram_hbm, idx_hbm, out_hbm, idx_v, *bufsems):
        bufs, sems = bufsems[:_NBUF], bufsems[_NBUF:]
        wid = lax.axis_index("s") * _NC + lax.axis_index("c")
        base = wid * bpw

        def unit_src(u):
            c, h = u // 2, u % 2
            return gram_hbm.at[idx_v.at[pl.ds(c * _CH, _CH)],
                               pl.ds(h * _HALFROW, _HALFROW)]

        def unit_dst(u):
            c, h = u // 2, u % 2
            return out_hbm.at[pl.ds(base + c * _CH, _CH),
                              pl.ds(h * _HALFROW, _HALFROW)]

        pltpu.sync_copy(idx_hbm.at[pl.ds(base, bpw)], idx_v)
        copies = [pltpu.async_copy(unit_src(u), bufs[u], sems[u])
                  for u in range(min(_NBUF, n_units))]
        for u in range(n_units):
            copies[u].wait()
            pltpu.sync_copy(bufs[u % _NBUF], unit_dst(u))
            if u + _NBUF < n_units:
                copies.append(pltpu.async_copy(
                    unit_src(u + _NBUF),
                    bufs[(u + _NBUF) % _NBUF], sems[(u + _NBUF) % _NBUF]))

    return _sc_gather


_MBLK = 128
_HGRID = HALF_N // _MBLK


def _dot_nt(a, b):
    return lax.dot_general(a, b, (((1,), (1,)), ((), ())),
                           preferred_element_type=jnp.float32)


def _dot_tn(a, b):
    return lax.dot_general(a, b, (((0,), (0,)), ((), ())),
                           preferred_element_type=jnp.float32)


def _tc_body(gath_ref, rate_ref, w1t_ref, b1c_ref, w2mt_ref, w2lt_ref,
             b2mc_ref, b2lc_ref, itT_ref, bi0_ref, zt_ref, out_ref, kl_ref,
             csc_ref):
    i = pl.program_id(0)

    @pl.when(i == 0)
    def _init():
        it = itT_ref[...]
        cn = jnp.sum(it * it, axis=0, keepdims=True)
        csc_ref[...] = 1.0 / jnp.maximum(jnp.sqrt(cn), 1e-12)
        kl_ref[...] = jnp.zeros((1, 1), jnp.float32)

    x = gath_ref[...] * (rate_ref[...] > 0).astype(jnp.float32)
    ss = jnp.sum(x * x, axis=1, keepdims=True)
    bi0 = x * (1.0 / jnp.maximum(jnp.sqrt(ss), 1e-12))
    bi0_ref[...] = bi0
    ht = jnp.tanh(_dot_nt(w1t_ref[...], bi0) + b1c_ref[...])
    meant = jnp.dot(w2mt_ref[...], ht,
                    preferred_element_type=jnp.float32) + b2mc_ref[...]
    logvart = jnp.dot(w2lt_ref[...], ht,
                      preferred_element_type=jnp.float32) + b2lc_ref[...]
    zt_ref[:, pl.ds(i * _MBLK, _MBLK)] = meant
    kl_ref[...] += jnp.sum(
        meant * meant + jnp.exp(logvart) - 1.0 - logvart).reshape(1, 1)
    zs = jnp.sum(meant * meant, axis=0, keepdims=True)
    znt = meant * (1.0 / jnp.maximum(jnp.sqrt(zs), 1e-12))
    out_ref[...] = (_dot_tn(znt, itT_ref[...])
                    * csc_ref[...]) * (1.0 / TAU_C)


def _tc_encoder(half, gathered, rating, W1T, b1c, W2mT, W2lT, b2mc, b2lc,
                itT, prev=None):
    off = half * _HGRID
    full = lambda shp: pl.BlockSpec(shp, lambda i: (0, 0))
    in_specs = [
        pl.BlockSpec((_MBLK, N_ITEMS), lambda i: (i, 0)),
        pl.BlockSpec((_MBLK, N_ITEMS), lambda i: (i + off, 0)),
        full((600, N_ITEMS)),
        full((600, 1)),
        full((LAT, 600)),
        full((LAT, 600)),
        full((LAT, 1)),
        full((LAT, 1)),
        full((LAT, N_ITEMS)),
    ]
    operands = [gathered, rating, W1T, b1c, W2mT, W2lT, b2mc, b2lc, itT]
    aliases = {}
    if prev is not None:
        in_specs += [full((8, 128)), full((8, 128))]
        operands += [prev[0], prev[1]]
        aliases = {9: 0, 10: 2}

    def body(*refs):
        if prev is not None:
            refs = refs[:9] + refs[11:]
        _tc_body(*refs)

    return pl.pallas_call(
        body,
        grid=(_HGRID,),
        in_specs=in_specs,
        out_specs=[
            pl.BlockSpec((_MBLK, N_ITEMS), lambda i: (i + off, 0)),
            pl.BlockSpec((LAT, HALF_N), lambda i: (0, 0)),
            pl.BlockSpec((_MBLK, N_ITEMS), lambda i: (i + off, 0)),
            pl.BlockSpec((1, 1), lambda i: (0, 0)),
        ],
        out_shape=[
            jax.ShapeDtypeStruct((BATCH_N, N_ITEMS), jnp.float32),
            jax.ShapeDtypeStruct((LAT, HALF_N), jnp.float32),
            jax.ShapeDtypeStruct((BATCH_N, N_ITEMS), jnp.float32),
            jax.ShapeDtypeStruct((1, 1), jnp.float32),
        ],
        scratch_shapes=[pltpu.VMEM((1, N_ITEMS), jnp.float32)],
        input_output_aliases=aliases,
        compiler_params=pltpu.CompilerParams(
            vmem_limit_bytes=100 * 1024 * 1024),
    )(*operands)


def kernel(rating_matrix_batch, rating_matrix_batch2, gram_matrix, W1, b1,
           W2, b2, items):
    idx = rating_matrix_batch2.astype(jnp.int32)
    gather = _make_sc_gather(HALF_N)
    g1 = gather(gram_matrix, idx[:HALF_N])
    g2 = gather(gram_matrix, idx[HALF_N:])

    W1T = W1.T
    itT = items.T
    b1c = b1.reshape(600, 1)
    W2T = W2.T
    W2mT = W2T[:LAT]
    W2lT = W2T[LAT:]
    b2mc = b2[:LAT].reshape(LAT, 1)
    b2lc = b2[LAT:].reshape(LAT, 1)
    wargs = (W1T, b1c, W2mT, W2lT, b2mc, b2lc, itT)

    bi0_1, zt1, out_1, kl1 = _tc_encoder(0, g1, rating_matrix_batch, *wargs)
    bi0, zt2, out, kl2 = _tc_encoder(1, g2, rating_matrix_batch, *wargs,
                                     prev=(bi0_1, out_1))
    z = jnp.concatenate([zt1, zt2], axis=1).T
    kl = 0.5 * (kl1[0, 0] + kl2[0, 0]) / BATCH_N
    return (z, out, kl, bi0)


# speedup vs baseline: 1.0114x; 1.0114x over previous
import functools

import jax
import jax.numpy as jnp
from jax import lax
from jax.experimental import pallas as pl
from jax.experimental.pallas import tpu as pltpu
from jax.experimental.pallas import tpu_sc as plsc

N_ITEMS = 8192
BATCH_N = 1024
SPLIT1 = 256
SPLIT2 = BATCH_N - SPLIT1
LAT = 200
TAU_C = 0.2

_NC = 2
_NS = 16
_NW = _NC * _NS
_CH = 8
_HALFROW = N_ITEMS // 2
_NBUF = 3


@functools.cache
def _make_sc_gather(rows):
    bpw = rows // _NW
    n_units = (bpw // _CH) * 2

    @functools.partial(
        pl.kernel,
        mesh=plsc.VectorSubcoreMesh(core_axis_name="c", subcore_axis_name="s"),
        out_type=jax.ShapeDtypeStruct((rows, N_ITEMS), jnp.float32),
        scratch_types=[
            pltpu.VMEM((bpw,), jnp.int32),
        ] + [pltpu.VMEM((_CH, _HALFROW), jnp.float32) for _ in range(_NBUF)]
          + [pltpu.SemaphoreType.DMA for _ in range(_NBUF)],
    )
    def _sc_gather(gram_hbm, idx_hbm, out_hbm, idx_v, *bufsems):
        bufs, sems = bufsems[:_NBUF], bufsems[_NBUF:]
        wid = lax.axis_index("s") * _NC + lax.axis_index("c")
        base = wid * bpw

        def unit_src(u):
            c, h = u // 2, u % 2
            return gram_hbm.at[idx_v.at[pl.ds(c * _CH, _CH)],
                               pl.ds(h * _HALFROW, _HALFROW)]

        def unit_dst(u):
            c, h = u // 2, u % 2
            return out_hbm.at[pl.ds(base + c * _CH, _CH),
                              pl.ds(h * _HALFROW, _HALFROW)]

        pltpu.sync_copy(idx_hbm.at[pl.ds(base, bpw)], idx_v)
        copies = [pltpu.async_copy(unit_src(u), bufs[u], sems[u])
                  for u in range(min(_NBUF, n_units))]
        for u in range(n_units):
            copies[u].wait()
            pltpu.sync_copy(bufs[u % _NBUF], unit_dst(u))
            if u + _NBUF < n_units:
                copies.append(pltpu.async_copy(
                    unit_src(u + _NBUF),
                    bufs[(u + _NBUF) % _NBUF], sems[(u + _NBUF) % _NBUF]))

    return _sc_gather


_MBLK = 128


def _dot_nt(a, b):
    return lax.dot_general(a, b, (((1,), (1,)), ((), ())),
                           preferred_element_type=jnp.float32)


def _dot_tn(a, b):
    return lax.dot_general(a, b, (((0,), (0,)), ((), ())),
                           preferred_element_type=jnp.float32)


def _tc_body(gath_ref, rate_ref, w1t_ref, b1c_ref, w2mt_ref, w2lt_ref,
             b2mc_ref, b2lc_ref, itT_ref, bi0_ref, zt_ref, out_ref, kl_ref,
             csc_ref):
    i = pl.program_id(0)

    @pl.when(i == 0)
    def _init():
        it = itT_ref[...]
        cn = jnp.sum(it * it, axis=0, keepdims=True)
        csc_ref[...] = 1.0 / jnp.maximum(jnp.sqrt(cn), 1e-12)
        kl_ref[...] = jnp.zeros((1, 1), jnp.float32)

    x = gath_ref[...] * (rate_ref[...] > 0).astype(jnp.float32)
    ss = jnp.sum(x * x, axis=1, keepdims=True)
    bi0 = x * (1.0 / jnp.maximum(jnp.sqrt(ss), 1e-12))
    bi0_ref[...] = bi0
    ht = jnp.tanh(_dot_nt(w1t_ref[...], bi0) + b1c_ref[...])
    meant = jnp.dot(w2mt_ref[...], ht,
                    preferred_element_type=jnp.float32) + b2mc_ref[...]
    logvart = jnp.dot(w2lt_ref[...], ht,
                      preferred_element_type=jnp.float32) + b2lc_ref[...]
    zt_ref[:, pl.ds(i * _MBLK, _MBLK)] = meant
    kl_ref[...] += jnp.sum(
        meant * meant + jnp.exp(logvart) - 1.0 - logvart).reshape(1, 1)
    zs = jnp.sum(meant * meant, axis=0, keepdims=True)
    znt = meant * (1.0 / jnp.maximum(jnp.sqrt(zs), 1e-12))
    out_ref[...] = (_dot_tn(znt, itT_ref[...])
                    * csc_ref[...]) * (1.0 / TAU_C)


def _tc_encoder(off, rows, gathered, rating, W1T, b1c, W2mT, W2lT, b2mc,
                b2lc, itT, prev=None):
    grid = rows // _MBLK
    full = lambda shp: pl.BlockSpec(shp, lambda i: (0, 0))
    in_specs = [
        pl.BlockSpec((_MBLK, N_ITEMS), lambda i: (i, 0)),
        pl.BlockSpec((_MBLK, N_ITEMS), lambda i: (i + off, 0)),
        full((600, N_ITEMS)),
        full((600, 1)),
        full((LAT, 600)),
        full((LAT, 600)),
        full((LAT, 1)),
        full((LAT, 1)),
        full((LAT, N_ITEMS)),
    ]
    operands = [gathered, rating, W1T, b1c, W2mT, W2lT, b2mc, b2lc, itT]
    aliases = {}
    if prev is not None:
        in_specs += [full((8, 128)), full((8, 128))]
        operands += [prev[0], prev[1]]
        aliases = {9: 0, 10: 2}

    def body(*refs):
        if prev is not None:
            refs = refs[:9] + refs[11:]
        _tc_body(*refs)

    return pl.pallas_call(
        body,
        grid=(grid,),
        in_specs=in_specs,
        out_specs=[
            pl.BlockSpec((_MBLK, N_ITEMS), lambda i: (i + off, 0)),
            pl.BlockSpec((LAT, rows), lambda i: (0, 0)),
            pl.BlockSpec((_MBLK, N_ITEMS), lambda i: (i + off, 0)),
            pl.BlockSpec((1, 1), lambda i: (0, 0)),
        ],
        out_shape=[
            jax.ShapeDtypeStruct((BATCH_N, N_ITEMS), jnp.float32),
            jax.ShapeDtypeStruct((LAT, rows), jnp.float32),
            jax.ShapeDtypeStruct((BATCH_N, N_ITEMS), jnp.float32),
            jax.ShapeDtypeStruct((1, 1), jnp.float32),
        ],
        scratch_shapes=[pltpu.VMEM((1, N_ITEMS), jnp.float32)],
        input_output_aliases=aliases,
        compiler_params=pltpu.CompilerParams(
            vmem_limit_bytes=100 * 1024 * 1024),
    )(*operands)


def kernel(rating_matrix_batch, rating_matrix_batch2, gram_matrix, W1, b1,
           W2, b2, items):
    idx = rating_matrix_batch2.astype(jnp.int32)
    g1 = _make_sc_gather(SPLIT1)(gram_matrix, idx[:SPLIT1])
    g2 = _make_sc_gather(SPLIT2)(gram_matrix, idx[SPLIT1:])

    W1T = W1.T
    itT = items.T
    b1c = b1.reshape(600, 1)
    W2T = W2.T
    W2mT = W2T[:LAT]
    W2lT = W2T[LAT:]
    b2mc = b2[:LAT].reshape(LAT, 1)
    b2lc = b2[LAT:].reshape(LAT, 1)
    wargs = (W1T, b1c, W2mT, W2lT, b2mc, b2lc, itT)

    bi0_1, zt1, out_1, kl1 = _tc_encoder(
        0, SPLIT1, g1, rating_matrix_batch, *wargs)
    bi0, zt2, out, kl2 = _tc_encoder(
        SPLIT1 // _MBLK, SPLIT2, g2, rating_matrix_batch, *wargs,
        prev=(bi0_1, out_1))
    z = jnp.concatenate([zt1, zt2], axis=1).T
    kl = 0.5 * (kl1[0, 0] + kl2[0, 0]) / BATCH_N
    return (z, out, kl, bi0)


# speedup vs baseline: 1.0982x; 1.0858x over previous
import functools

import jax
import jax.numpy as jnp
from jax import lax
from jax.experimental import pallas as pl
from jax.experimental.pallas import tpu as pltpu
from jax.experimental.pallas import tpu_sc as plsc

N_ITEMS = 8192
BATCH_N = 1024
LAT = 200
TAU_C = 0.2

_NC = 2
_NS = 16
_NW = _NC * _NS
_BPW = BATCH_N // _NW
_CH = 8
_HALFROW = N_ITEMS // 2
_NU = (_BPW // _CH) * 2
_NBUF = 3


@functools.cache
def _make_sc_gather():
    @functools.partial(
        pl.kernel,
        mesh=plsc.VectorSubcoreMesh(core_axis_name="c", subcore_axis_name="s"),
        out_type=jax.ShapeDtypeStruct((BATCH_N, N_ITEMS), jnp.float32),
        scratch_types=[
            pltpu.VMEM((_BPW,), jnp.int32),
        ] + [pltpu.VMEM((_CH, _HALFROW), jnp.float32) for _ in range(_NBUF)]
          + [pltpu.SemaphoreType.DMA for _ in range(_NBUF)],
    )
    def _sc_gather(gram_hbm, idx_hbm, out_hbm, idx_v, *bufsems):
        bufs, sems = bufsems[:_NBUF], bufsems[_NBUF:]
        wid = lax.axis_index("s") * _NC + lax.axis_index("c")
        base = wid * _BPW

        def unit_src(u):
            c, h = u // 2, u % 2
            return gram_hbm.at[idx_v.at[pl.ds(c * _CH, _CH)],
                               pl.ds(h * _HALFROW, _HALFROW)]

        def unit_dst(u):
            c, h = u // 2, u % 2
            return out_hbm.at[pl.ds(base + c * _CH, _CH),
                              pl.ds(h * _HALFROW, _HALFROW)]

        pltpu.sync_copy(idx_hbm.at[pl.ds(base, _BPW)], idx_v)
        copies = [pltpu.async_copy(unit_src(u), bufs[u], sems[u])
                  for u in range(_NBUF)]
        for u in range(_NU):
            copies[u].wait()
            pltpu.sync_copy(bufs[u % _NBUF], unit_dst(u))
            if u + _NBUF < _NU:
                copies.append(pltpu.async_copy(
                    unit_src(u + _NBUF),
                    bufs[(u + _NBUF) % _NBUF], sems[(u + _NBUF) % _NBUF]))

    return _sc_gather


_MBLK = 128
_GRID = BATCH_N // _MBLK


def _dot_nt(a, b):
    return lax.dot_general(a, b, (((1,), (1,)), ((), ())),
                           preferred_element_type=jnp.float32)


def _dot_tn(a, b):
    return lax.dot_general(a, b, (((0,), (0,)), ((), ())),
                           preferred_element_type=jnp.float32)


def _tc_body(gath_ref, rate_ref, w1t_ref, b1c_ref, w2mt_ref, w2lt_ref,
             b2mc_ref, b2lc_ref, itT_ref, bi0_ref, zt_ref, out_ref, kl_ref,
             csc_ref):
    i = pl.program_id(0)

    @pl.when(i == 0)
    def _init():
        it = itT_ref[...]
        cn = jnp.sum(it * it, axis=0, keepdims=True)
        csc_ref[...] = 1.0 / jnp.maximum(jnp.sqrt(cn), 1e-12)
        kl_ref[...] = jnp.zeros((1, 1), jnp.float32)

    x = gath_ref[...] * (rate_ref[...] > 0).astype(jnp.float32)
    ss = jnp.sum(x * x, axis=1, keepdims=True)
    bi0 = x * (1.0 / jnp.maximum(jnp.sqrt(ss), 1e-12))
    bi0_ref[...] = bi0
    ht = jnp.tanh(_dot_nt(w1t_ref[...], bi0) + b1c_ref[...])
    meant = jnp.dot(w2mt_ref[...], ht,
                    preferred_element_type=jnp.float32) + b2mc_ref[...]
    logvart = jnp.dot(w2lt_ref[...], ht,
                      preferred_element_type=jnp.float32) + b2lc_ref[...]
    zt_ref[:, pl.ds(i * _MBLK, _MBLK)] = meant
    kl_ref[...] += jnp.sum(
        meant * meant + jnp.exp(logvart) - 1.0 - logvart).reshape(1, 1)
    zs = jnp.sum(meant * meant, axis=0, keepdims=True)
    znt = meant * (1.0 / jnp.maximum(jnp.sqrt(zs), 1e-12))
    out_ref[...] = (_dot_tn(znt, itT_ref[...])
                    * csc_ref[...]) * (1.0 / TAU_C)


def _tc_encoder(gathered, rating, W1T, b1c, W2mT, W2lT, b2mc, b2lc, itT):
    full = lambda shp: pl.BlockSpec(shp, lambda i: (0, 0))
    blk = lambda shp: pl.BlockSpec(shp, lambda i: (i, 0))
    return pl.pallas_call(
        _tc_body,
        grid=(_GRID,),
        in_specs=[
            blk((_MBLK, N_ITEMS)),
            blk((_MBLK, N_ITEMS)),
            full((600, N_ITEMS)),
            full((600, 1)),
            full((LAT, 600)),
            full((LAT, 600)),
            full((LAT, 1)),
            full((LAT, 1)),
            full((LAT, N_ITEMS)),
        ],
        out_specs=[
            blk((_MBLK, N_ITEMS)),
            pl.BlockSpec((LAT, BATCH_N), lambda i: (0, 0)),
            blk((_MBLK, N_ITEMS)),
            pl.BlockSpec((1, 1), lambda i: (0, 0)),
        ],
        out_shape=[
            jax.ShapeDtypeStruct((BATCH_N, N_ITEMS), jnp.float32),
            jax.ShapeDtypeStruct((LAT, BATCH_N), jnp.float32),
            jax.ShapeDtypeStruct((BATCH_N, N_ITEMS), jnp.float32),
            jax.ShapeDtypeStruct((1, 1), jnp.float32),
        ],
        scratch_shapes=[pltpu.VMEM((1, N_ITEMS), jnp.float32)],
        compiler_params=pltpu.CompilerParams(
            vmem_limit_bytes=100 * 1024 * 1024),
    )(gathered, rating, W1T, b1c, W2mT, W2lT, b2mc, b2lc, itT)


def kernel(rating_matrix_batch, rating_matrix_batch2, gram_matrix, W1, b1,
           W2, b2, items):
    idx = rating_matrix_batch2.astype(jnp.int32)
    gathered = _make_sc_gather()(gram_matrix, idx)

    W1T = W1.T
    itT = items.T
    b1c = b1.reshape(600, 1)
    W2T = W2.T
    W2mT = W2T[:LAT]
    W2lT = W2T[LAT:]
    b2mc = b2[:LAT].reshape(LAT, 1)
    b2lc = b2[LAT:].reshape(LAT, 1)

    bi0, zt_p, out, klp = _tc_encoder(
        gathered, rating_matrix_batch, W1T, b1c, W2mT, W2lT, b2mc, b2lc, itT)
    z = zt_p.T
    kl = 0.5 * klp[0, 0] / BATCH_N
    return (z, out, kl, bi0)
